# baseline (speedup 1.0000x reference)
import jax
import jax.numpy as jnp
from jax import lax
from jax.experimental import pallas as pl
from jax.experimental.pallas import tpu as pltpu
from jax.experimental.pallas import tpu_sc as plsc

_N, _B, _D = 2048, 4, 128
_NE = 2050

_TC_N = 1536
_SC_N = _N - _TC_N
_BN = 512

_NW = 32
_IW = _SC_N // _NW


def _tc_body(inc_ref, x_ref, tab_ref, out_ref):
    counts_t = jnp.sum(inc_ref[...], axis=-1).T
    iota_ne = lax.broadcasted_iota(jnp.int32, (1, _NE), 1)
    tab = tab_ref[...].astype(jnp.bfloat16)
    for b in range(_B):
        lvl = counts_t[:, b:b + 1] + 1
        oh = (lvl == iota_ne).astype(jnp.bfloat16)
        emb = jnp.dot(oh, tab, preferred_element_type=jnp.float32)
        out_ref[:, b, :] = x_ref[:, b, :] + emb


def _tc_part(node_incidences, x, pos_embedding):
    return pl.pallas_call(
        _tc_body,
        grid=(_TC_N // _BN,),
        in_specs=[
            pl.BlockSpec((_B, _BN, _N), lambda n: (0, n, 0)),
            pl.BlockSpec((_BN, _B, _D), lambda n: (n, 0, 0)),
            pl.BlockSpec((_NE, _D), lambda n: (0, 0)),
        ],
        out_specs=pl.BlockSpec((_BN, _B, _D), lambda n: (n, 0, 0)),
        out_shape=jax.ShapeDtypeStruct((_N, _B, _D), jnp.float32),
    )(node_incidences, x, pos_embedding)


def _sc_body(x_hbm, inc_hbm, tab_hbm, out_hbm,
             buf0, buf1, idx_v, gat_v, x_v,
             sem0, sem1, sem_g, sem_x):
    wid = lax.axis_index("s") * 2 + lax.axis_index("c")
    i0 = _TC_N + wid * _IW
    lanes = lax.iota(jnp.int32, 16)

    cp_x = []
    for b in range(_B):
        cp_x.append(pltpu.async_copy(
            x_hbm.at[pl.ds(i0, _IW), b], x_v.at[b], sem_x))

    bufs, sems = (buf0, buf1), (sem0, sem1)

    def _start(b):
        src = inc_hbm.at[pl.ds(b * _N + i0, _IW)]
        return pltpu.async_copy(src, bufs[b % 2], sems[b % 2])

    cp_g = []
    cps = {0: _start(0)}
    for b in range(_B):
        if b + 1 < _B:
            cps[b + 1] = _start(b + 1)
        cps[b].wait()
        buf = bufs[b % 2]

        def _row(di, vec, buf=buf):
            zeros = jnp.zeros((16,), jnp.int32)
            accs = [zeros, zeros, zeros, zeros]
            for col in range(_N // 16):
                m = buf[di, pl.ds(col * 16, 16)] == 1
                accs[col % 4] = accs[col % 4] + plsc.all_reduce_population_count(m)
            lvl = (accs[0] + accs[1]) + (accs[2] + accs[3]) + 1
            return jnp.where(lanes == di, lvl, vec)

        idx_v[b, :] = lax.fori_loop(0, _IW, _row, jnp.zeros((16,), jnp.int32))
        cp_g.append(pltpu.async_copy(
            tab_hbm.at[idx_v.at[b]], gat_v.at[b], sem_g))

    for cp in cp_x + cp_g:
        cp.wait()

    def _addrow(r, carry):
        for b in range(_B):
            for cc in range(_D // 16):
                s = pl.ds(cc * 16, 16)
                gat_v[b, r, s] = gat_v[b, r, s] + x_v[b, r, s]
        return carry

    lax.fori_loop(0, _IW, _addrow, 0)
    for b in range(_B):
        pltpu.sync_copy(gat_v.at[b], out_hbm.at[pl.ds(wid * _IW, _IW), b])


def _sc_part(x, inc_flat, table):
    mesh = plsc.VectorSubcoreMesh(core_axis_name="c", subcore_axis_name="s")
    f = pl.kernel(
        _sc_body,
        mesh=mesh,
        compiler_params=pltpu.CompilerParams(needs_layout_passes=False),
        out_type=jax.ShapeDtypeStruct((_SC_N, _B, _D), jnp.float32),
        scratch_types=[
            pltpu.VMEM((_IW, _N), jnp.int32),
            pltpu.VMEM((_IW, _N), jnp.int32),
            pltpu.VMEM((_B, _IW), jnp.int32),
            pltpu.VMEM((_B, _IW, _D), jnp.float32),
            pltpu.VMEM((_B, _IW, _D), jnp.float32),
            pltpu.SemaphoreType.DMA,
            pltpu.SemaphoreType.DMA,
            pltpu.SemaphoreType.DMA,
            pltpu.SemaphoreType.DMA,
        ],
    )
    return f(x, inc_flat, table)


def kernel(x, node_incidences, pos_embedding):
    out_tc = _tc_part(node_incidences, x, pos_embedding)
    inc_flat = node_incidences.reshape(_B * _N, _N)
    out_sc = _sc_part(x, inc_flat, pos_embedding)
    return lax.dynamic_update_slice(out_tc, out_sc, (_TC_N, 0, 0))


# speedup vs baseline: 1.6768x; 1.6768x over previous
import jax
import jax.numpy as jnp
from jax import lax
from jax.experimental import pallas as pl
from jax.experimental.pallas import tpu as pltpu

_N, _B, _D = 2048, 4, 128
_NE = 2050
_BN = 512


def _body(inc_ref, x_ref, tab_ref, out_ref):
    counts_t = jnp.sum(inc_ref[...], axis=-1).T
    iota_ne = lax.broadcasted_iota(jnp.int32, (1, _NE), 1)
    tab = tab_ref[...].astype(jnp.bfloat16)
    for b in range(_B):
        lvl = counts_t[:, b:b + 1] + 1
        oh = (lvl == iota_ne).astype(jnp.bfloat16)
        emb = jnp.dot(oh, tab, preferred_element_type=jnp.float32)
        out_ref[:, b, :] = x_ref[:, b, :] + emb


def kernel(x, node_incidences, pos_embedding):
    return pl.pallas_call(
        _body,
        grid=(_N // _BN,),
        in_specs=[
            pl.BlockSpec((_B, _BN, _N), lambda n: (0, n, 0)),
            pl.BlockSpec((_BN, _B, _D), lambda n: (n, 0, 0)),
            pl.BlockSpec((_NE, _D), lambda n: (0, 0)),
        ],
        out_specs=pl.BlockSpec((_BN, _B, _D), lambda n: (n, 0, 0)),
        out_shape=jax.ShapeDtypeStruct((_N, _B, _D), jnp.float32),
    )(node_incidences, x, pos_embedding)
